# baseline (speedup 1.0000x reference)
import jax
import jax.numpy as jnp
from jax.experimental import pallas as pl
from jax.experimental.pallas import tpu as pltpu

F32 = jnp.float32
B = 256
R = 2048
EPS = 1e-5

_SIG = (2, 0, 1)


def _dot(a, b, ca, cb):
    return jax.lax.dot_general(a, b, (((ca,), (cb,)), ((), ())),
                               preferred_element_type=F32)


def _onehot(ids):
    return (jax.lax.broadcasted_iota(jnp.int32, (B, ids.shape[1]), 0)
            == ids).astype(F32)


def _head_epilogue(acc, s, out_ref):
    s0 = jnp.maximum(s[0:1, :], 1e-30)
    s1 = jnp.maximum(s[1:2, :], 1e-30)
    s2 = jnp.maximum(s[2:3, :], 1e-30)

    h10 = [acc[k:k + 1, :] / s1 for k in range(3)]
    h01 = [acc[3 + k:4 + k, :] / s1 for k in range(3)]
    h11 = [acc[12 + k:13 + k, :] / s0 for k in range(9)]

    rvec = [h11[3 * _SIG[i] + _SIG[j]] for i in range(3) for j in range(3)]
    for k in range(9):
        out_ref[12 + k:13 + k, :] = rvec[k]

    norm2 = sum(r * r for r in rvec)
    norm = jnp.maximum(jnp.sqrt(norm2), 1e-5)
    rv = [r / norm for r in rvec]
    M = [[rv[3 * d + c] for d in range(3)] for c in range(3)]

    S = {}
    for i in range(3):
        for j in range(i, 3):
            S[(i, j)] = sum(M[c][i] * M[c][j] for c in range(3))
    one = jnp.ones_like(S[(0, 0)])
    zero = jnp.zeros_like(one)
    V = [[one if i == j else zero for j in range(3)] for i in range(3)]

    def sget(i, j):
        return S[(i, j)] if i <= j else S[(j, i)]

    def sset(i, j, v):
        S[(min(i, j), max(i, j))] = v

    for _ in range(6):
        for (p, q) in ((0, 1), (0, 2), (1, 2)):
            app, aqq, apq = sget(p, p), sget(q, q), sget(p, q)
            small = jnp.abs(apq) < 1e-30
            apq_s = jnp.where(small, one, apq)
            tau = (aqq - app) / (2.0 * apq_s)
            sgn = jnp.where(tau >= 0, one, -one)
            t = sgn / (jnp.abs(tau) + jnp.sqrt(1.0 + tau * tau))
            t = jnp.where(small, zero, t)
            c = jax.lax.rsqrt(1.0 + t * t)
            s_ = t * c
            r = 3 - p - q
            spr, sqr = sget(p, r), sget(q, r)
            sset(p, r, c * spr - s_ * sqr)
            sset(q, r, s_ * spr + c * sqr)
            sset(p, p, app - t * apq)
            sset(q, q, aqq + t * apq)
            sset(p, q, zero)
            for i in range(3):
                vip, viq = V[i][p], V[i][q]
                V[i][p] = c * vip - s_ * viq
                V[i][q] = s_ * vip + c * viq

    d = [sget(0, 0), sget(1, 1), sget(2, 2)]
    for (a, bcol) in ((0, 1), (0, 2), (1, 2)):
        sw = d[a] < d[bcol]
        d[a], d[bcol] = (jnp.where(sw, d[bcol], d[a]),
                         jnp.where(sw, d[a], d[bcol]))
        for i in range(3):
            va, vb = V[i][a], V[i][bcol]
            V[i][a] = jnp.where(sw, vb, va)
            V[i][bcol] = jnp.where(sw, va, vb)

    def matvec(col):
        return [sum(M[c][k] * V[k][col] for k in range(3)) for c in range(3)]

    u1 = matvec(0)
    n1 = jnp.sqrt(sum(u * u for u in u1))
    u1 = [u / jnp.maximum(n1, 1e-20) for u in u1]
    u2 = matvec(1)
    proj = sum(a_ * b_ for a_, b_ in zip(u1, u2))
    u2 = [u - proj * v for u, v in zip(u2, u1)]
    n2 = jnp.sqrt(sum(u * u for u in u2))
    u2 = [u / jnp.maximum(n2, 1e-20) for u in u2]
    u3 = [u1[1] * u2[2] - u1[2] * u2[1],
          u1[2] * u2[0] - u1[0] * u2[2],
          u1[0] * u2[1] - u1[1] * u2[0]]
    detV = (V[0][0] * (V[1][1] * V[2][2] - V[1][2] * V[2][1])
            - V[0][1] * (V[1][0] * V[2][2] - V[1][2] * V[2][0])
            + V[0][2] * (V[1][0] * V[2][1] - V[1][1] * V[2][0]))

    Rm = [[u1[c] * V[dd][0] + u2[c] * V[dd][1] + detV * u3[c] * V[dd][2]
           for dd in range(3)] for c in range(3)]
    for c in range(3):
        for dd in range(3):
            out_ref[3 * c + dd:3 * c + dd + 1, :] = Rm[c][dd]

    m1 = [acc[6 + k:7 + k, :] / s2 for k in range(3)]
    m2 = [acc[9 + k:10 + k, :] / s2 for k in range(3)]
    tb = [h01[_SIG[i]] for i in range(3)]
    ta = [h10[_SIG[i]] for i in range(3)]
    for c in range(3):
        tv = m2[c] + tb[c] - sum(Rm[c][dd] * (m1[dd] + ta[dd])
                                 for dd in range(3))
        out_ref[9 + c:10 + c, :] = tv


def _stats(xt, ids3, alpha_c, gamma_c, K):
    def body(x_ref, ids_ref, alpha_ref, gamma_ref, scale_ref, stat_ref):
        i = pl.program_id(0)
        oh = _onehot(ids_ref[0])
        x = x_ref[...]
        vals = jnp.concatenate([x, x * x, jnp.ones((8, R), F32)], axis=0)
        blk = _dot(vals, oh, 1, 1)

        @pl.when(i == 0)
        def _():
            stat_ref[...] = blk

        @pl.when(i > 0)
        def _():
            stat_ref[...] += blk

        @pl.when(i == K - 1)
        def _():
            acc = stat_ref[...]
            inv = 1.0 / jnp.maximum(acc[64:65, :], 1.0)
            mean = acc[0:32, :] * inv
            ex2 = acc[32:64, :] * inv
            a = alpha_ref[...]
            var = ex2 - (2.0 * a - a * a) * mean * mean
            scale_ref[...] = gamma_ref[...] * jax.lax.rsqrt(var + EPS)

    return pl.pallas_call(
        body,
        grid=(K,),
        in_specs=[
            pl.BlockSpec((32, R), lambda i: (0, i)),
            pl.BlockSpec((1, 1, R), lambda i: (i, 0, 0)),
            pl.BlockSpec((32, 1), lambda i: (0, 0)),
            pl.BlockSpec((32, 1), lambda i: (0, 0)),
        ],
        out_specs=pl.BlockSpec((32, B), lambda i: (0, 0)),
        out_shape=jax.ShapeDtypeStruct((32, B), F32),
        scratch_shapes=[pltpu.VMEM((72, B), F32)],
    )(xt, ids3, alpha_c, gamma_c)


def _main(xt, ids3, streams, scale, lin_w, K):
    def body(x_ref, ids_ref, st_ref, scale_ref, w_ref,
             out_ref, s_ref, a_ref):
        j = pl.program_id(0)
        oh = _onehot(ids_ref[0])
        sc = _dot(scale_ref[...], oh, 1, 0)
        logits = _dot(w_ref[...], x_ref[...] * sc, 1, 0)
        eb = jnp.exp(logits)
        e8 = jnp.concatenate([eb, jnp.zeros((5, R), F32)], axis=0)
        bs = _dot(e8, oh, 1, 1)
        wrows = jnp.concatenate([
            jnp.broadcast_to(eb[1:2, :], (6, R)),
            jnp.broadcast_to(eb[2:3, :], (6, R)),
            jnp.broadcast_to(eb[0:1, :], (9, R)),
            jnp.zeros((3, R), F32),
        ], axis=0)
        bstr = _dot(st_ref[...] * wrows, oh, 1, 1)

        @pl.when(j == 0)
        def _():
            s_ref[...] = bs
            a_ref[...] = bstr

        @pl.when(j > 0)
        def _():
            s_ref[...] += bs
            a_ref[...] += bstr

        @pl.when(j == K - 1)
        def _():
            _head_epilogue(a_ref[...], s_ref[...], out_ref)

    full = lambda i: (0, 0)
    return pl.pallas_call(
        body,
        grid=(K,),
        in_specs=[
            pl.BlockSpec((32, R), lambda i: (0, i)),
            pl.BlockSpec((1, 1, R), lambda i: (i, 0, 0)),
            pl.BlockSpec((24, R), lambda i: (0, i)),
            pl.BlockSpec((32, B), full),
            pl.BlockSpec((3, 32), full),
        ],
        out_specs=pl.BlockSpec((32, B), full),
        out_shape=jax.ShapeDtypeStruct((32, B), F32),
        scratch_shapes=[
            pltpu.VMEM((8, B), F32),
            pltpu.VMEM((24, B), F32),
        ],
    )(xt, ids3, streams, scale, lin_w)


def kernel(x00, x10, x01, x11, pos, segment_ids, gn_gamma, gn_beta,
           gn_alpha, lin_w, lin_b, W10, W01, W11):
    del gn_beta, lin_b
    N = x00.shape[0]
    NP = -(-N // R) * R
    K = NP // R
    P = NP - N

    def padt(a):
        return jnp.pad(a, ((0, 0), (0, P))) if P else a

    xt = padt(x00[:, :, 0].T)
    ids_p = jnp.pad(segment_ids.astype(jnp.int32), (0, P),
                    constant_values=B)
    ids3 = ids_p.reshape(K, 1, R)
    alpha_c = gn_alpha.reshape(32, 1)
    gamma_c = gn_gamma.reshape(32, 1)
    x10m = jnp.einsum('h,nhd->dn', W10[0], x10)
    x01m = jnp.einsum('h,nhd->dn', W01[0], x01)
    x11m = jnp.einsum('h,nhd->dn', W11[0], x11)
    streams = padt(jnp.concatenate(
        [x10m, x01m, pos.T, x11m,
         jnp.zeros((3, N), F32)], axis=0))

    scale = _stats(xt, ids3, alpha_c, gamma_c, K)
    out32 = _main(xt, ids3, streams, scale, lin_w, K)

    rot = out32[0:9].T.reshape(B, 3, 3)
    t = out32[9:12].T
    r_vector = out32[12:21].T
    return rot, t, r_vector
